# baseline (speedup 1.0000x reference)
import functools

import jax
import jax.numpy as jnp
from jax import lax
from jax.experimental import pallas as pl
from jax.experimental.pallas import tpu as pltpu
from jax.experimental.pallas import tpu_sc as plsc

_LANES = 16
_ROWS = 8
_NSLOTS = 4
_AHEAD = 2


def kernel(inputs, embeddings):
    B, S, D = inputs.shape
    assert D % _LANES == 0

    info = plsc.get_sparse_core_info()
    NC, NS = info.num_cores, info.num_subcores
    NW = NC * NS

    rows_per_worker = S // NW
    n_chunks = rows_per_worker // _ROWS
    assert S % NW == 0 and rows_per_worker % _ROWS == 0
    assert n_chunks % _NSLOTS == 0
    n_groups = n_chunks // _NSLOTS
    n_col_vecs = D // _LANES

    mesh = plsc.VectorSubcoreMesh(core_axis_name="c", subcore_axis_name="s")

    @functools.partial(
        pl.kernel,
        mesh=mesh,
        out_type=jax.ShapeDtypeStruct((B, S, D), jnp.float32),
        scratch_types=[
            pltpu.VMEM((_NSLOTS, _ROWS, D), jnp.float32),
            pltpu.VMEM((_NSLOTS, B, _ROWS, D), jnp.float32),
            pltpu.SemaphoreType.DMA,
            pltpu.SemaphoreType.DMA,
            pltpu.SemaphoreType.DMA,
            pltpu.SemaphoreType.DMA,
            pltpu.SemaphoreType.DMA,
            pltpu.SemaphoreType.DMA,
            pltpu.SemaphoreType.DMA,
            pltpu.SemaphoreType.DMA,
        ],
    )
    def k(x_hbm, e_hbm, o_hbm, emb_v, in_v,
          si0, si1, si2, si3, so0, so1, so2, so3):
        sem_in = (si0, si1, si2, si3)
        sem_out = (so0, so1, so2, so3)
        wid = lax.axis_index("s") * NC + lax.axis_index("c")
        base_row = wid * rows_per_worker

        def in_copies(c, slot):
            r0 = base_row + c * _ROWS
            cps = [pltpu.make_async_copy(
                e_hbm.at[pl.ds(r0, _ROWS)], emb_v.at[slot], sem_in[slot])]
            for b in range(B):
                cps.append(pltpu.make_async_copy(
                    x_hbm.at[b, pl.ds(r0, _ROWS)],
                    in_v.at[slot, b], sem_in[slot]))
            return cps

        def out_copy(c, slot, b):
            r0 = base_row + c * _ROWS
            return pltpu.make_async_copy(
                in_v.at[slot, b], o_hbm.at[b, pl.ds(r0, _ROWS)],
                sem_out[slot])

        for c0 in range(_AHEAD):
            for cp in in_copies(c0, c0):
                cp.start()

        def group_body(it, carry):
            i0 = it * _NSLOTS
            for j in range(_NSLOTS):
                c = i0 + j
                nslot = (j + _AHEAD) % _NSLOTS

                if j + _AHEAD >= _NSLOTS:
                    for b in range(B):
                        out_copy(c + _AHEAD - _NSLOTS, nslot, b).wait()

                    @pl.when(it < n_groups - 1)
                    def _():
                        for cp in in_copies(c + _AHEAD, nslot):
                            cp.start()
                else:
                    @pl.when(c + _AHEAD - _NSLOTS >= 0)
                    def _():
                        for b in range(B):
                            out_copy(c + _AHEAD - _NSLOTS, nslot, b).wait()
                    for cp in in_copies(c + _AHEAD, nslot):
                        cp.start()

                for cp in in_copies(c, j):
                    cp.wait()

                for b0 in range(0, B, 2):
                    @plsc.parallel_loop(0, _ROWS, unroll=2)
                    def add_body(row, _b0=b0, _j=j):
                        for u in range(n_col_vecs):
                            sl = pl.ds(u * _LANES, _LANES)
                            e = emb_v[_j, row, sl]
                            plsc.addupdate(in_v.at[_j, _b0, row, sl], e)
                            plsc.addupdate(in_v.at[_j, _b0 + 1, row, sl], e)

                    out_copy(c, j, b0).start()
                    out_copy(c, j, b0 + 1).start()
            return carry

        lax.fori_loop(0, n_groups, group_body, 0)

        for d in range(_NSLOTS - _AHEAD):
            c = n_chunks - (_NSLOTS - _AHEAD) + d
            for b in range(B):
                out_copy(c, c % _NSLOTS, b).wait()

    return k(inputs, embeddings)


# speedup vs baseline: 1.1729x; 1.1729x over previous
import functools

import jax
import jax.numpy as jnp
from jax import lax
from jax.experimental import pallas as pl
from jax.experimental.pallas import tpu as pltpu
from jax.experimental.pallas import tpu_sc as plsc

_LANES = 16
_ROWS = 8
_NSLOTS = 4
_AHEAD = 2


def kernel(inputs, embeddings):
    B, S, D = inputs.shape
    assert D % _LANES == 0

    info = plsc.get_sparse_core_info()
    NC, NS = info.num_cores, info.num_subcores
    NW = NC * NS

    rows_per_worker = S // NW
    n_chunks = rows_per_worker // _ROWS
    assert S % NW == 0 and rows_per_worker % _ROWS == 0
    assert n_chunks % _NSLOTS == 0
    n_groups = n_chunks // _NSLOTS
    n_col_vecs = D // _LANES

    mesh = plsc.VectorSubcoreMesh(core_axis_name="c", subcore_axis_name="s")

    @functools.partial(
        pl.kernel,
        mesh=mesh,
        out_type=jax.ShapeDtypeStruct((B, S, D), jnp.float32),
        scratch_types=[
            pltpu.VMEM((_NSLOTS, _ROWS, D), jnp.float32),
            pltpu.VMEM((_NSLOTS, B, _ROWS, D), jnp.float32),
            pltpu.SemaphoreType.DMA,
            pltpu.SemaphoreType.DMA,
            pltpu.SemaphoreType.DMA,
            pltpu.SemaphoreType.DMA,
            pltpu.SemaphoreType.DMA,
            pltpu.SemaphoreType.DMA,
            pltpu.SemaphoreType.DMA,
            pltpu.SemaphoreType.DMA,
        ],
    )
    def k(x_hbm, e_hbm, o_hbm, emb_v, in_v,
          si0, si1, si2, si3, so0, so1, so2, so3):
        sem_in = (si0, si1, si2, si3)
        sem_out = (so0, so1, so2, so3)
        wid = lax.axis_index("s") * NC + lax.axis_index("c")
        base_row = wid * rows_per_worker

        def in_copies(c, slot):
            r0 = base_row + c * _ROWS
            cps = [pltpu.make_async_copy(
                e_hbm.at[pl.ds(r0, _ROWS)], emb_v.at[slot], sem_in[slot])]
            for b in range(B):
                cps.append(pltpu.make_async_copy(
                    x_hbm.at[b, pl.ds(r0, _ROWS)],
                    in_v.at[slot, b], sem_in[slot]))
            return cps

        def out_copy(c, slot, b):
            r0 = base_row + c * _ROWS
            return pltpu.make_async_copy(
                in_v.at[slot, b], o_hbm.at[b, pl.ds(r0, _ROWS)],
                sem_out[slot])

        for c0 in range(_AHEAD):
            for cp in in_copies(c0, c0):
                cp.start()

        def group_body(it, carry):
            i0 = it * _NSLOTS
            for j in range(_NSLOTS):
                c = i0 + j
                nslot = (j + _AHEAD) % _NSLOTS

                if j + _AHEAD >= _NSLOTS:
                    for b in range(B):
                        out_copy(c + _AHEAD - _NSLOTS, nslot, b).wait()

                    @pl.when(it < n_groups - 1)
                    def _():
                        for cp in in_copies(c + _AHEAD, nslot):
                            cp.start()
                else:
                    @pl.when(c + _AHEAD - _NSLOTS >= 0)
                    def _():
                        for b in range(B):
                            out_copy(c + _AHEAD - _NSLOTS, nslot, b).wait()
                    for cp in in_copies(c + _AHEAD, nslot):
                        cp.start()

                for cp in in_copies(c, j):
                    cp.wait()

                for b0 in range(0, B, 2):
                    @plsc.parallel_loop(0, _ROWS, unroll=1)
                    def add_body(row, _b0=b0, _j=j):
                        for u in range(n_col_vecs):
                            sl = pl.ds(u * _LANES, _LANES)
                            e = emb_v[_j, row, sl]
                            plsc.addupdate(in_v.at[_j, _b0, row, sl], e)
                            plsc.addupdate(in_v.at[_j, _b0 + 1, row, sl], e)

                    out_copy(c, j, b0).start()
                    out_copy(c, j, b0 + 1).start()
            return carry

        lax.fori_loop(0, n_groups, group_body, 0)

        for d in range(_NSLOTS - _AHEAD):
            c = n_chunks - (_NSLOTS - _AHEAD) + d
            for b in range(B):
                out_copy(c, c % _NSLOTS, b).wait()

    return k(inputs, embeddings)
